# baseline (speedup 1.0000x reference)
import functools

import jax
import jax.numpy as jnp
from jax import lax
from jax.experimental import pallas as pl
from jax.experimental.pallas import tpu as pltpu
from jax.experimental.pallas import tpu_sc as plsc

_D = 128
_K = 4
_NBUF = 4
_PF = 2


@functools.lru_cache(maxsize=None)
def _make_gather(nw, nc, phase_base, bp, hist):
    b_per_w = bp // nw
    n_grp = b_per_w // _K
    mesh = plsc.VectorSubcoreMesh(core_axis_name="c", subcore_axis_name="s")

    @functools.partial(
        pl.kernel,
        mesh=mesh,
        out_type=jax.ShapeDtypeStruct((bp, hist, _D), jnp.float32),
        scratch_types=[
            pltpu.VMEM((b_per_w, hist), jnp.int32),
            pltpu.VMEM((_NBUF, _K, hist, _D), jnp.float32),
        ]
        + [pltpu.SemaphoreType.DMA] * (2 * _NBUF),
    )
    def k(idx_hbm, table_hbm, out_hbm, idx_v, rows_v, *sems):
        gsem, ssem = sems[:_NBUF], sems[_NBUF:]
        wid = lax.axis_index("s") * nc + lax.axis_index("c")
        bat0 = wid * b_per_w

        def start_gather(c, b):
            for j in range(_K):
                pltpu.async_copy(
                    table_hbm.at[idx_v.at[c * _K + j]],
                    rows_v.at[b, j],
                    gsem[b],
                )

        def wait_gather(b):
            for j in range(_K):
                pltpu.make_async_copy(
                    table_hbm.at[idx_v.at[0]], rows_v.at[b, j], gsem[b]
                ).wait()

        def start_store(c, b):
            pltpu.async_copy(
                rows_v.at[b],
                out_hbm.at[pl.ds(bat0 + c * _K, _K)],
                ssem[b],
            )

        def wait_store(b):
            pltpu.make_async_copy(
                rows_v.at[b],
                out_hbm.at[pl.ds(bat0, _K)],
                ssem[b],
            ).wait()

        pltpu.sync_copy(idx_hbm.at[pl.ds(phase_base + bat0, b_per_w)], idx_v)

        for c in range(_PF):
            start_gather(c, c % _NBUF)

        def consume(c, b, prime, store_wait):
            if prime:
                bp = (b + _PF) % _NBUF
                if store_wait:
                    wait_store(bp)
                start_gather(c + _PF, bp)
            wait_gather(b)
            start_store(c, b)

        for c in range(_NBUF):
            consume(c, c % _NBUF, prime=True, store_wait=(c + _PF >= _NBUF))

        n_outer = n_grp // _NBUF

        def outer(g, carry):
            for b in range(_NBUF):
                consume(g * _NBUF + b, b, prime=True, store_wait=True)
            return carry

        lax.fori_loop(1, n_outer - 1, outer, 0)

        for c in range(n_grp - _NBUF, n_grp):
            consume(c, c % _NBUF, prime=(c + _PF < n_grp), store_wait=True)

        for b in range(_NBUF):
            wait_store(b)

    return k


_PHASES = 2


def kernel(x, table):
    batch, hist = x.shape
    info = plsc.get_sparse_core_info()
    nc, ns = info.num_cores, info.num_subcores
    nw = nc * ns
    bp = batch // _PHASES
    assert bp % (nw * _K * _NBUF) == 0
    xi = x.astype(jnp.int32)
    outs = [
        _make_gather(nw, nc, p * bp, bp, hist)(xi, table)
        for p in range(_PHASES)
    ]
    return jnp.concatenate(outs, axis=0)


# speedup vs baseline: 1.6064x; 1.6064x over previous
import functools

import jax
import jax.numpy as jnp
from jax import lax
from jax.experimental import pallas as pl
from jax.experimental.pallas import tpu as pltpu
from jax.experimental.pallas import tpu_sc as plsc

_D = 128
_K = 4
_NBUF = 4
_PF = 2


@functools.lru_cache(maxsize=None)
def _make_gather(nw, nc, phase_base, bp, hist):
    b_per_w = bp // nw
    n_grp = b_per_w // _K
    mesh = plsc.VectorSubcoreMesh(core_axis_name="c", subcore_axis_name="s")

    @functools.partial(
        pl.kernel,
        mesh=mesh,
        out_type=jax.ShapeDtypeStruct((bp, hist, _D), jnp.float32),
        scratch_types=[
            pltpu.VMEM((b_per_w, hist), jnp.int32),
            pltpu.VMEM((_NBUF, _K, hist, _D), jnp.float32),
        ]
        + [pltpu.SemaphoreType.DMA] * (2 * _NBUF),
    )
    def k(idx_hbm, table_hbm, out_hbm, idx_v, rows_v, *sems):
        gsem, ssem = sems[:_NBUF], sems[_NBUF:]
        wid = lax.axis_index("s") * nc + lax.axis_index("c")
        bat0 = wid * b_per_w

        def start_gather(c, b):
            for j in range(_K):
                pltpu.async_copy(
                    table_hbm.at[idx_v.at[c * _K + j]],
                    rows_v.at[b, j],
                    gsem[b],
                )

        def wait_gather(b):
            for j in range(_K):
                pltpu.make_async_copy(
                    table_hbm.at[idx_v.at[0]], rows_v.at[b, j], gsem[b]
                ).wait()

        def start_store(c, b):
            pltpu.async_copy(
                rows_v.at[b],
                out_hbm.at[pl.ds(bat0 + c * _K, _K)],
                ssem[b],
            )

        def wait_store(b):
            pltpu.make_async_copy(
                rows_v.at[b],
                out_hbm.at[pl.ds(bat0, _K)],
                ssem[b],
            ).wait()

        pltpu.sync_copy(idx_hbm.at[pl.ds(phase_base + bat0, b_per_w)], idx_v)

        for c in range(_PF):
            start_gather(c, c % _NBUF)

        def consume(c, b, prime, store_wait):
            if prime:
                bp = (b + _PF) % _NBUF
                if store_wait:
                    wait_store(bp)
                start_gather(c + _PF, bp)
            wait_gather(b)
            start_store(c, b)

        for c in range(_NBUF):
            consume(c, c % _NBUF, prime=True, store_wait=(c + _PF >= _NBUF))

        n_outer = n_grp // _NBUF

        def outer(g, carry):
            for b in range(_NBUF):
                consume(g * _NBUF + b, b, prime=True, store_wait=True)
            return carry

        lax.fori_loop(1, n_outer - 1, outer, 0)

        for c in range(n_grp - _NBUF, n_grp):
            consume(c, c % _NBUF, prime=(c + _PF < n_grp), store_wait=True)

        for b in range(_NBUF):
            wait_store(b)

    return k


def kernel(x, table):
    batch, hist = x.shape
    info = plsc.get_sparse_core_info()
    nc, ns = info.num_cores, info.num_subcores
    nw = nc * ns
    assert batch % (nw * _K * _NBUF) == 0
    return _make_gather(nw, nc, 0, batch, hist)(x.astype(jnp.int32), table)
